# baseline (speedup 1.0000x reference)
import functools

import jax
import jax.numpy as jnp
from jax import lax
from jax.experimental import pallas as pl
from jax.experimental.pallas import tpu as pltpu
from jax.experimental.pallas import tpu_sc as plsc

N = 10000
EMB = 256
K = 1024
G = 128
NC = 10
CW = 1.0

BN = 1000
NREAL = N // BN
NBLK = 13

NTILES = 32
CHUNKS = 3
CB = 128
PER_TILE = CHUNKS * CB
N2 = NTILES * PER_TILE
N3 = NBLK * BN
GK = G * K
STRIPE = 2 * GK // 16
ZB = 2048


def _split3(v):
    h = v.astype(jnp.bfloat16)
    r = v - h.astype(jnp.float32)
    mid = r.astype(jnp.bfloat16)
    low = (r - mid.astype(jnp.float32)).astype(jnp.bfloat16)
    return h, mid, low


def _split2(v):
    h = v.astype(jnp.bfloat16)
    mid = (v - h.astype(jnp.float32)).astype(jnp.bfloat16)
    return h, mid


def _stage_a_body(x_ref, sc_ref, bt_ref, cb_ref,
                  fl_ref, sw_ref, vw_ref, p_ref, q_ref, cnt_ref, loss_ref):
    pid = pl.program_id(0)

    @pl.when(pid == 0)
    def _init():
        p_ref[...] = jnp.zeros_like(p_ref)
        q_ref[...] = jnp.zeros_like(q_ref)
        cnt_ref[...] = jnp.zeros_like(cnt_ref)
        loss_ref[...] = jnp.zeros_like(loss_ref)

    @pl.when(pid >= NREAL)
    def _pad():
        fl_ref[...] = jnp.zeros_like(fl_ref)
        sw_ref[...] = jnp.zeros_like(sw_ref)
        vw_ref[...] = jnp.zeros_like(vw_ref)

    @pl.when(pid < NREAL)
    def _real():
        x = x_ref[...]
        cb = cb_ref[...]
        xc = lax.dot_general(x, cb, (((1,), (1,)), ((), ())),
                             preferred_element_type=jnp.float32)
        x2 = jnp.sum(x * x, axis=1, keepdims=True)
        csh, csm, csl = _split3(cb * cb)
        ones_row = jnp.ones((1, EMB), jnp.bfloat16)

        def odot(rhs):
            return lax.dot_general(ones_row, rhs, (((1,), (1,)), ((), ())),
                                   preferred_element_type=jnp.float32)

        c2 = odot(csl) + odot(csm) + odot(csh)
        d = x2 - 2.0 * xc + c2
        m = jnp.min(d, axis=1, keepdims=True)
        kio = lax.broadcasted_iota(jnp.int32, (BN, K), 1)
        a = jnp.min(jnp.where(d == m, kio, K), axis=1, keepdims=True)

        score = sc_ref[...]
        bt = bt_ref[...]
        fl_ref[...] = bt * K + a
        sw_ref[...] = score
        vw_ref[...] = jnp.ones_like(score)

        ohb = (lax.broadcasted_iota(jnp.int32, (BN, G), 1)
               == bt).astype(jnp.bfloat16)

        def tdot(lhs, rhs):
            return lax.dot_general(lhs, rhs, (((0,), (0,)), ((), ())),
                                   preferred_element_type=jnp.float32)

        xh, xm = _split2(x)
        sh, sm = _split2(score)
        p_ref[...] += (tdot(ohb * sm, xh) + tdot(ohb * sh, xm)
                       + tdot(ohb * sh, xh))
        q_ref[...] += tdot(ohb, xm) + tdot(ohb, xh)
        cnt_ref[...] += tdot(ohb, jnp.ones((BN, 1), jnp.bfloat16))
        loss_ref[...] += jnp.sum(m).reshape(1, 1)


_stage_a = pl.pallas_call(
    _stage_a_body,
    grid=(NBLK,),
    in_specs=[
        pl.BlockSpec((BN, EMB), lambda i: (jnp.minimum(i, NREAL - 1), 0)),
        pl.BlockSpec((BN, 1), lambda i: (jnp.minimum(i, NREAL - 1), 0)),
        pl.BlockSpec((BN, 1), lambda i: (jnp.minimum(i, NREAL - 1), 0)),
        pl.BlockSpec((K, EMB), lambda i: (0, 0)),
    ],
    out_specs=[
        pl.BlockSpec((BN, 1), lambda i: (i, 0)),
        pl.BlockSpec((BN, 1), lambda i: (i, 0)),
        pl.BlockSpec((BN, 1), lambda i: (i, 0)),
        pl.BlockSpec((G, EMB), lambda i: (0, 0)),
        pl.BlockSpec((G, EMB), lambda i: (0, 0)),
        pl.BlockSpec((G, 1), lambda i: (0, 0)),
        pl.BlockSpec((1, 1), lambda i: (0, 0)),
    ],
    out_shape=[
        jax.ShapeDtypeStruct((N3, 1), jnp.int32),
        jax.ShapeDtypeStruct((N3, 1), jnp.float32),
        jax.ShapeDtypeStruct((N3, 1), jnp.float32),
        jax.ShapeDtypeStruct((G, EMB), jnp.float32),
        jax.ShapeDtypeStruct((G, EMB), jnp.float32),
        jax.ShapeDtypeStruct((G, 1), jnp.float32),
        jax.ShapeDtypeStruct((1, 1), jnp.float32),
    ],
    compiler_params=pltpu.CompilerParams(dimension_semantics=("arbitrary",)),
)


def _sc_stage_body(fl_hbm, sw_hbm, vw_hbm, out_hbm,
                   fi, fi2, sw, vw, zbuf, acc):
    cid = lax.axis_index("c")
    sid = lax.axis_index("s")
    tid = cid * 16 + sid
    for jj in range(ZB // 16):
        zbuf[pl.ds(jj * 16, 16)] = jnp.zeros((16,), jnp.float32)
    for kk in range(STRIPE // ZB):
        pltpu.sync_copy(zbuf, acc.at[pl.ds(sid * STRIPE + kk * ZB, ZB)])
    pltpu.sync_copy(fl_hbm.at[tid], fi)
    pltpu.sync_copy(sw_hbm.at[tid], sw)
    pltpu.sync_copy(vw_hbm.at[tid], vw)
    for j in range(CHUNKS):
        for l in range(CB // 16):
            s_ = pl.ds(l * 16, 16)
            fi2[j, s_] = fi[j, s_] + GK
    plsc.subcore_barrier()
    for j in range(CHUNKS):
        pltpu.sync_copy(sw.at[j], acc.at[fi.at[j]], add=True)
        pltpu.sync_copy(vw.at[j], acc.at[fi2.at[j]], add=True)
    plsc.subcore_barrier()
    pltpu.sync_copy(acc.at[pl.ds(sid * STRIPE, STRIPE)],
                    out_hbm.at[cid, pl.ds(sid * STRIPE, STRIPE)])


@functools.cache
def _build_sc_stage():
    return functools.partial(
        pl.kernel,
        mesh=plsc.VectorSubcoreMesh(core_axis_name="c", subcore_axis_name="s"),
        out_type=jax.ShapeDtypeStruct((2, 2 * GK), jnp.float32),
        scratch_types=[
            pltpu.VMEM((CHUNKS, CB), jnp.int32),
            pltpu.VMEM((CHUNKS, CB), jnp.int32),
            pltpu.VMEM((CHUNKS, CB), jnp.float32),
            pltpu.VMEM((CHUNKS, CB), jnp.float32),
            pltpu.VMEM((ZB,), jnp.float32),
            pltpu.VMEM_SHARED((2 * GK,), jnp.float32),
        ],
    )(_sc_stage_body)


def _stage_c_body(a_ref, cb_ref, p_ref, q_ref, cnt_ref, loss_ref, w_ref, b_ref,
                  logit_ref, cg_ref, sg_ref, lo_ref):
    a_s = a_ref[0, 0] + a_ref[1, 0]
    a_n = a_ref[0, 1] + a_ref[1, 1]
    cb = cb_ref[...]

    def ndot(lhs, rhs):
        return lax.dot_general(lhs, rhs, (((1,), (0,)), ((), ())),
                               preferred_element_type=jnp.float32)

    ch, cm = _split2(cb)
    sh, sm = _split2(a_s)
    nh, nm = _split2(a_n)
    r = ndot(sm, ch) + ndot(sh, cm) + ndot(sh, ch)
    s = ndot(nm, ch) + ndot(nh, cm) + ndot(nh, ch)
    cnt = jnp.maximum(cnt_ref[...], 1.0)
    p = p_ref[...]
    cr = p + r
    cg = cr / cnt
    sg = (q_ref[...] + s - cr) / cnt
    cg_ref[...] = cg
    sg_ref[...] = sg
    logit_ref[...] = lax.dot_general(cg, w_ref[...], (((1,), (0,)), ((), ())),
                                     preferred_element_type=jnp.float32) + b_ref[...]
    lo_ref[...] = loss_ref[...] * (CW / (N * EMB))


_stage_c = pl.pallas_call(
    _stage_c_body,
    out_shape=[
        jax.ShapeDtypeStruct((G, NC), jnp.float32),
        jax.ShapeDtypeStruct((G, EMB), jnp.float32),
        jax.ShapeDtypeStruct((G, EMB), jnp.float32),
        jax.ShapeDtypeStruct((1, 1), jnp.float32),
    ],
)


def kernel(node_feat, score, batch, codebook, W, b):
    batch = batch.astype(jnp.int32)

    fl, sw, vw, p_sum, q_sum, cnt, loss = _stage_a(
        node_feat, score, batch[:, None], codebook)

    a_mats = _build_sc_stage()(fl[:N2, 0].reshape(NTILES, CHUNKS, CB),
                               sw[:N2, 0].reshape(NTILES, CHUNKS, CB),
                               vw[:N2, 0].reshape(NTILES, CHUNKS, CB))

    logit, c_graph, s_graph, lo = _stage_c(
        a_mats.reshape(2, 2, G, K), codebook, p_sum, q_sum, cnt, loss,
        W, b[None, :])
    return (logit, c_graph, s_graph, lo[0, 0])


# speedup vs baseline: 1.2340x; 1.2340x over previous
import functools

import jax
import jax.numpy as jnp
from jax import lax
from jax.experimental import pallas as pl
from jax.experimental.pallas import tpu as pltpu
from jax.experimental.pallas import tpu_sc as plsc

N = 10000
EMB = 256
K = 1024
G = 128
NC = 10
CW = 1.0

BN = 1000
NBLK = N // BN

NTILES = 32
CHUNKS = 3
CB = 128
PER_TILE = CHUNKS * CB
N2 = NTILES * PER_TILE
GK = G * K
STRIPE = 2 * GK // 16
ZB = 2048


def _split3(v):
    h = v.astype(jnp.bfloat16)
    r = v - h.astype(jnp.float32)
    mid = r.astype(jnp.bfloat16)
    low = (r - mid.astype(jnp.float32)).astype(jnp.bfloat16)
    return h, mid, low


def _split2(v):
    h = v.astype(jnp.bfloat16)
    mid = (v - h.astype(jnp.float32)).astype(jnp.bfloat16)
    return h, mid


def _stage_a_body(x_ref, sc_ref, bt_ref, cb_ref,
                  idx_ref, p_ref, q_ref, cnt_ref, loss_ref, c2_ref):
    pid = pl.program_id(0)

    @pl.when(pid == 0)
    def _init():
        p_ref[...] = jnp.zeros_like(p_ref)
        q_ref[...] = jnp.zeros_like(q_ref)
        cnt_ref[...] = jnp.zeros_like(cnt_ref)
        loss_ref[...] = jnp.zeros_like(loss_ref)
        csh, csm, csl = _split3(cb_ref[...] * cb_ref[...])
        ones_row = jnp.ones((1, EMB), jnp.bfloat16)

        def odot(rhs):
            return lax.dot_general(ones_row, rhs, (((1,), (1,)), ((), ())),
                                   preferred_element_type=jnp.float32)

        c2_ref[...] = odot(csl) + odot(csm) + odot(csh)

    x = x_ref[...]
    cb = cb_ref[...]
    xc = lax.dot_general(x, cb, (((1,), (1,)), ((), ())),
                         preferred_element_type=jnp.float32)
    x2 = jnp.sum(x * x, axis=1, keepdims=True)
    d = x2 - 2.0 * xc + c2_ref[...]
    m = jnp.min(d, axis=1, keepdims=True)
    kio = lax.broadcasted_iota(jnp.int32, (BN, K), 1)
    a = jnp.min(jnp.where(d == m, kio, K), axis=1, keepdims=True)

    score = sc_ref[...]
    bt = bt_ref[...]
    idx_ref[...] = bt * K + a

    ohb = (lax.broadcasted_iota(jnp.int32, (BN, G), 1)
           == bt).astype(jnp.bfloat16)

    def tdot(lhs, rhs):
        return lax.dot_general(lhs, rhs, (((0,), (0,)), ((), ())),
                               preferred_element_type=jnp.float32)

    xh, xm = _split2(x)
    sh, sm = _split2(score)
    p_ref[...] += (tdot(ohb * sm, xh) + tdot(ohb * sh, xm)
                   + tdot(ohb * sh, xh))
    q_ref[...] += tdot(ohb, xm) + tdot(ohb, xh)
    cnt_ref[...] += tdot(ohb, jnp.ones((BN, 1), jnp.bfloat16))
    loss_ref[...] += jnp.sum(m).reshape(1, 1)


_stage_a = pl.pallas_call(
    _stage_a_body,
    grid=(NBLK,),
    in_specs=[
        pl.BlockSpec((BN, EMB), lambda i: (i, 0)),
        pl.BlockSpec((BN, 1), lambda i: (i, 0)),
        pl.BlockSpec((BN, 1), lambda i: (i, 0)),
        pl.BlockSpec((K, EMB), lambda i: (0, 0)),
    ],
    out_specs=[
        pl.BlockSpec((BN, 1), lambda i: (i, 0)),
        pl.BlockSpec((G, EMB), lambda i: (0, 0)),
        pl.BlockSpec((G, EMB), lambda i: (0, 0)),
        pl.BlockSpec((G, 1), lambda i: (0, 0)),
        pl.BlockSpec((1, 1), lambda i: (0, 0)),
    ],
    out_shape=[
        jax.ShapeDtypeStruct((N, 1), jnp.int32),
        jax.ShapeDtypeStruct((G, EMB), jnp.float32),
        jax.ShapeDtypeStruct((G, EMB), jnp.float32),
        jax.ShapeDtypeStruct((G, 1), jnp.float32),
        jax.ShapeDtypeStruct((1, 1), jnp.float32),
    ],
    scratch_shapes=[pltpu.VMEM((1, K), jnp.float32)],
    compiler_params=pltpu.CompilerParams(dimension_semantics=("arbitrary",)),
)


def _sc_stage_body(fl_hbm, sw_hbm, vw_hbm, out_hbm,
                   fi, fi2, sw, vw, zbuf, acc, sem, sem2):
    cid = lax.axis_index("c")
    sid = lax.axis_index("s")
    tid = cid * 16 + sid
    ld1 = pltpu.async_copy(fl_hbm.at[tid], fi, sem)
    ld2 = pltpu.async_copy(sw_hbm.at[tid], sw, sem)
    ld3 = pltpu.async_copy(vw_hbm.at[tid], vw, sem)
    for jj in range(ZB // 16):
        zbuf[pl.ds(jj * 16, 16)] = jnp.zeros((16,), jnp.float32)
    zc = [pltpu.async_copy(zbuf, acc.at[pl.ds(sid * STRIPE + kk * ZB, ZB)],
                           sem2)
          for kk in range(STRIPE // ZB)]
    ld1.wait()
    ld2.wait()
    ld3.wait()
    for j in range(CHUNKS):
        for l in range(CB // 16):
            s_ = pl.ds(l * 16, 16)
            fi2[j, s_] = fi[j, s_] + GK
    for c in zc:
        c.wait()
    plsc.subcore_barrier()
    scs = []
    for j in range(CHUNKS):
        scs.append(pltpu.async_copy(sw.at[j], acc.at[fi.at[j]], sem, add=True))
        scs.append(pltpu.async_copy(vw.at[j], acc.at[fi2.at[j]], sem,
                                    add=True))
    for c in scs:
        c.wait()
    plsc.subcore_barrier()
    pltpu.sync_copy(acc.at[pl.ds(sid * STRIPE, STRIPE)],
                    out_hbm.at[cid, pl.ds(sid * STRIPE, STRIPE)])


@functools.cache
def _build_sc_stage():
    return functools.partial(
        pl.kernel,
        mesh=plsc.VectorSubcoreMesh(core_axis_name="c", subcore_axis_name="s"),
        out_type=jax.ShapeDtypeStruct((2, 2 * GK), jnp.float32),
        scratch_types=[
            pltpu.VMEM((CHUNKS, CB), jnp.int32),
            pltpu.VMEM((CHUNKS, CB), jnp.int32),
            pltpu.VMEM((CHUNKS, CB), jnp.float32),
            pltpu.VMEM((CHUNKS, CB), jnp.float32),
            pltpu.VMEM((ZB,), jnp.float32),
            pltpu.VMEM_SHARED((2 * GK,), jnp.float32),
            pltpu.SemaphoreType.DMA,
            pltpu.SemaphoreType.DMA,
        ],
    )(_sc_stage_body)


def _stage_c_body(a_ref, cb_ref, p_ref, q_ref, cnt_ref, loss_ref, w_ref, b_ref,
                  logit_ref, cg_ref, sg_ref, lo_ref):
    a_s = a_ref[0, 0] + a_ref[1, 0]
    a_n = a_ref[0, 1] + a_ref[1, 1]
    cb = cb_ref[...]

    def ndot(lhs, rhs):
        return lax.dot_general(lhs, rhs, (((1,), (0,)), ((), ())),
                               preferred_element_type=jnp.float32)

    ch, cm = _split2(cb)
    sh, sm = _split2(a_s)
    nh, nm = _split2(a_n)
    r = ndot(sm, ch) + ndot(sh, cm) + ndot(sh, ch)
    s = ndot(nm, ch) + ndot(nh, cm) + ndot(nh, ch)
    cnt = jnp.maximum(cnt_ref[...], 1.0)
    p = p_ref[...]
    cr = p + r
    cg = cr / cnt
    sg = (q_ref[...] + s - cr) / cnt
    cg_ref[...] = cg
    sg_ref[...] = sg
    logit_ref[...] = lax.dot_general(cg, w_ref[...], (((1,), (0,)), ((), ())),
                                     preferred_element_type=jnp.float32) + b_ref[...]
    lo_ref[...] = loss_ref[...] * (CW / (N * EMB))


_stage_c = pl.pallas_call(
    _stage_c_body,
    out_shape=[
        jax.ShapeDtypeStruct((G, NC), jnp.float32),
        jax.ShapeDtypeStruct((G, EMB), jnp.float32),
        jax.ShapeDtypeStruct((G, EMB), jnp.float32),
        jax.ShapeDtypeStruct((1, 1), jnp.float32),
    ],
)


def kernel(node_feat, score, batch, codebook, W, b):
    batch = batch.astype(jnp.int32)

    fl, p_sum, q_sum, cnt, loss = _stage_a(
        node_feat, score, batch[:, None], codebook)

    fl2 = jnp.zeros((N2,), jnp.int32).at[:N].set(fl[:, 0])
    sw2 = jnp.zeros((N2,), jnp.float32).at[:N].set(score[:, 0])
    vw2 = jnp.zeros((N2,), jnp.float32).at[:N].set(1.0)
    a_mats = _build_sc_stage()(fl2.reshape(NTILES, CHUNKS, CB),
                               sw2.reshape(NTILES, CHUNKS, CB),
                               vw2.reshape(NTILES, CHUNKS, CB))

    logit, c_graph, s_graph, lo = _stage_c(
        a_mats.reshape(2, 2, G, K), codebook, p_sum, q_sum, cnt, loss,
        W, b[None, :])
    return (logit, c_graph, s_graph, lo[0, 0])


# speedup vs baseline: 1.2452x; 1.0091x over previous
import functools

import jax
import jax.numpy as jnp
from jax import lax
from jax.experimental import pallas as pl
from jax.experimental.pallas import tpu as pltpu
from jax.experimental.pallas import tpu_sc as plsc

N = 10000
EMB = 256
K = 1024
G = 128
NC = 10
CW = 1.0

BN = 1000
NBLK = N // BN

NTILES = 32
CHUNKS = 3
CB = 128
PER_TILE = CHUNKS * CB
N2 = NTILES * PER_TILE
GK = G * K
STRIPE = 2 * GK // 16
ZB = 2048


def _split3(v):
    h = v.astype(jnp.bfloat16)
    r = v - h.astype(jnp.float32)
    mid = r.astype(jnp.bfloat16)
    low = (r - mid.astype(jnp.float32)).astype(jnp.bfloat16)
    return h, mid, low


def _split2(v):
    h = v.astype(jnp.bfloat16)
    mid = (v - h.astype(jnp.float32)).astype(jnp.bfloat16)
    return h, mid


def _stage_a_body(x_ref, sc_ref, bt_ref, cb_ref,
                  idx_ref, p_ref, q_ref, cnt_ref, loss_ref, c2_ref):
    pid = pl.program_id(0)

    @pl.when(pid == 0)
    def _init():
        p_ref[...] = jnp.zeros_like(p_ref)
        q_ref[...] = jnp.zeros_like(q_ref)
        cnt_ref[...] = jnp.zeros_like(cnt_ref)
        loss_ref[...] = jnp.zeros_like(loss_ref)
        csh, csm, csl = _split3(cb_ref[...] * cb_ref[...])
        ones_row = jnp.ones((1, EMB), jnp.bfloat16)

        def odot(rhs):
            return lax.dot_general(ones_row, rhs, (((1,), (1,)), ((), ())),
                                   preferred_element_type=jnp.float32)

        c2_ref[...] = odot(csl) + odot(csm) + odot(csh)

    x = x_ref[...]
    cb = cb_ref[...]
    xc = lax.dot_general(x, cb, (((1,), (1,)), ((), ())),
                         preferred_element_type=jnp.float32)
    x2 = jnp.sum(x * x, axis=1, keepdims=True)
    d = x2 - 2.0 * xc + c2_ref[...]
    m = jnp.min(d, axis=1, keepdims=True)
    kio = lax.broadcasted_iota(jnp.int32, (BN, K), 1)
    a = jnp.min(jnp.where(d == m, kio, K), axis=1, keepdims=True)

    score = sc_ref[...]
    bt = bt_ref[...]
    idx_ref[...] = bt * K + a

    ohb = (lax.broadcasted_iota(jnp.int32, (BN, G), 1)
           == bt).astype(jnp.bfloat16)

    def tdot(lhs, rhs):
        return lax.dot_general(lhs, rhs, (((0,), (0,)), ((), ())),
                               preferred_element_type=jnp.float32)

    xh, xm = _split2(x)
    sh, sm = _split2(score)
    p_ref[...] += (tdot(ohb * sm, xh) + tdot(ohb * sh, xm)
                   + tdot(ohb * sh, xh))
    q_ref[...] += tdot(ohb, xm) + tdot(ohb, xh)
    cnt_ref[...] += tdot(ohb, jnp.ones((BN, 1), jnp.bfloat16))
    loss_ref[...] += jnp.sum(m).reshape(1, 1)


_stage_a = pl.pallas_call(
    _stage_a_body,
    grid=(NBLK,),
    in_specs=[
        pl.BlockSpec((BN, EMB), lambda i: (i, 0)),
        pl.BlockSpec((BN, 1), lambda i: (i, 0)),
        pl.BlockSpec((BN, 1), lambda i: (i, 0)),
        pl.BlockSpec((K, EMB), lambda i: (0, 0)),
    ],
    out_specs=[
        pl.BlockSpec((BN, 1), lambda i: (i, 0)),
        pl.BlockSpec((G, EMB), lambda i: (0, 0)),
        pl.BlockSpec((G, EMB), lambda i: (0, 0)),
        pl.BlockSpec((G, 1), lambda i: (0, 0)),
        pl.BlockSpec((1, 1), lambda i: (0, 0)),
    ],
    out_shape=[
        jax.ShapeDtypeStruct((N, 1), jnp.int32),
        jax.ShapeDtypeStruct((G, EMB), jnp.float32),
        jax.ShapeDtypeStruct((G, EMB), jnp.float32),
        jax.ShapeDtypeStruct((G, 1), jnp.float32),
        jax.ShapeDtypeStruct((1, 1), jnp.float32),
    ],
    scratch_shapes=[pltpu.VMEM((1, K), jnp.float32)],
    compiler_params=pltpu.CompilerParams(dimension_semantics=("arbitrary",)),
)


def _sc_stage_body(fl_hbm, sw_hbm, vw_hbm, out_hbm,
                   fi, fi2, sw, vw, zbuf, acc, sem, sem2):
    cid = lax.axis_index("c")
    sid = lax.axis_index("s")
    tid = cid * 16 + sid
    ld1 = pltpu.async_copy(fl_hbm.at[tid], fi, sem)
    ld2 = pltpu.async_copy(sw_hbm.at[tid], sw, sem)
    ld3 = pltpu.async_copy(vw_hbm.at[tid], vw, sem)
    for jj in range(ZB // 16):
        zbuf[pl.ds(jj * 16, 16)] = jnp.zeros((16,), jnp.float32)
    zc = [pltpu.async_copy(zbuf, acc.at[pl.ds(sid * STRIPE + kk * ZB, ZB)],
                           sem2)
          for kk in range(STRIPE // ZB)]
    ld1.wait()
    ld2.wait()
    ld3.wait()
    for j in range(CHUNKS):
        for l in range(CB // 16):
            s_ = pl.ds(l * 16, 16)
            fi2[j, s_] = fi[j, s_] + GK
    for c in zc:
        c.wait()
    plsc.subcore_barrier()
    for j in range(CHUNKS):
        c1 = pltpu.async_copy(sw.at[j], acc.at[fi.at[j]], sem, add=True)
        c2 = pltpu.async_copy(vw.at[j], acc.at[fi2.at[j]], sem2, add=True)
        c1.wait()
        c2.wait()
    plsc.subcore_barrier()
    pltpu.sync_copy(acc.at[pl.ds(sid * STRIPE, STRIPE)],
                    out_hbm.at[cid, pl.ds(sid * STRIPE, STRIPE)])


@functools.cache
def _build_sc_stage():
    return functools.partial(
        pl.kernel,
        mesh=plsc.VectorSubcoreMesh(core_axis_name="c", subcore_axis_name="s"),
        out_type=jax.ShapeDtypeStruct((2, 2 * GK), jnp.float32),
        scratch_types=[
            pltpu.VMEM((CHUNKS, CB), jnp.int32),
            pltpu.VMEM((CHUNKS, CB), jnp.int32),
            pltpu.VMEM((CHUNKS, CB), jnp.float32),
            pltpu.VMEM((CHUNKS, CB), jnp.float32),
            pltpu.VMEM((ZB,), jnp.float32),
            pltpu.VMEM_SHARED((2 * GK,), jnp.float32),
            pltpu.SemaphoreType.DMA,
            pltpu.SemaphoreType.DMA,
        ],
    )(_sc_stage_body)


def _stage_c_body(a_ref, cb_ref, p_ref, q_ref, cnt_ref, loss_ref, w_ref, b_ref,
                  logit_ref, cg_ref, sg_ref, lo_ref):
    a_s = a_ref[0, 0] + a_ref[1, 0]
    a_n = a_ref[0, 1] + a_ref[1, 1]
    cb = cb_ref[...]

    def ndot(lhs, rhs):
        return lax.dot_general(lhs, rhs, (((1,), (0,)), ((), ())),
                               preferred_element_type=jnp.float32)

    ch, cm = _split2(cb)
    sh, sm = _split2(a_s)
    nh, nm = _split2(a_n)
    r = ndot(sm, ch) + ndot(sh, cm) + ndot(sh, ch)
    s = ndot(nm, ch) + ndot(nh, cm) + ndot(nh, ch)
    cnt = jnp.maximum(cnt_ref[...], 1.0)
    p = p_ref[...]
    cr = p + r
    cg = cr / cnt
    sg = (q_ref[...] + s - cr) / cnt
    cg_ref[...] = cg
    sg_ref[...] = sg
    logit_ref[...] = lax.dot_general(cg, w_ref[...], (((1,), (0,)), ((), ())),
                                     preferred_element_type=jnp.float32) + b_ref[...]
    lo_ref[...] = loss_ref[...] * (CW / (N * EMB))


_stage_c = pl.pallas_call(
    _stage_c_body,
    out_shape=[
        jax.ShapeDtypeStruct((G, NC), jnp.float32),
        jax.ShapeDtypeStruct((G, EMB), jnp.float32),
        jax.ShapeDtypeStruct((G, EMB), jnp.float32),
        jax.ShapeDtypeStruct((1, 1), jnp.float32),
    ],
)


def kernel(node_feat, score, batch, codebook, W, b):
    batch = batch.astype(jnp.int32)

    fl, p_sum, q_sum, cnt, loss = _stage_a(
        node_feat, score, batch[:, None], codebook)

    fl2 = jnp.zeros((N2,), jnp.int32).at[:N].set(fl[:, 0])
    sw2 = jnp.zeros((N2,), jnp.float32).at[:N].set(score[:, 0])
    vw2 = jnp.zeros((N2,), jnp.float32).at[:N].set(1.0)
    a_mats = _build_sc_stage()(fl2.reshape(NTILES, CHUNKS, CB),
                               sw2.reshape(NTILES, CHUNKS, CB),
                               vw2.reshape(NTILES, CHUNKS, CB))

    logit, c_graph, s_graph, lo = _stage_c(
        a_mats.reshape(2, 2, G, K), codebook, p_sum, q_sum, cnt, loss,
        W, b[None, :])
    return (logit, c_graph, s_graph, lo[0, 0])


# speedup vs baseline: 1.2497x; 1.0036x over previous
import functools

import jax
import jax.numpy as jnp
from jax import lax
from jax.experimental import pallas as pl
from jax.experimental.pallas import tpu as pltpu
from jax.experimental.pallas import tpu_sc as plsc

N = 10000
EMB = 256
K = 1024
G = 128
NC = 10
CW = 1.0

BN = 1000
NBLK = N // BN

NTILES = 32
CHUNKS = 3
CB = 128
PER_TILE = CHUNKS * CB
N2 = NTILES * PER_TILE
GK = G * K
STRIPE = 2 * GK // 16
ZB = 2048


def _split3(v):
    h = v.astype(jnp.bfloat16)
    r = v - h.astype(jnp.float32)
    mid = r.astype(jnp.bfloat16)
    low = (r - mid.astype(jnp.float32)).astype(jnp.bfloat16)
    return h, mid, low


def _split2(v):
    h = v.astype(jnp.bfloat16)
    mid = (v - h.astype(jnp.float32)).astype(jnp.bfloat16)
    return h, mid


def _stage_a1_body(x_ref, bt_ref, cb_ref, idx_ref, loss_ref, c2_ref):
    pid = pl.program_id(0)

    @pl.when(pid == 0)
    def _init():
        loss_ref[...] = jnp.zeros_like(loss_ref)
        csh, csm, csl = _split3(cb_ref[...] * cb_ref[...])
        ones_row = jnp.ones((1, EMB), jnp.bfloat16)

        def odot(rhs):
            return lax.dot_general(ones_row, rhs, (((1,), (1,)), ((), ())),
                                   preferred_element_type=jnp.float32)

        c2_ref[...] = odot(csl) + odot(csm) + odot(csh)

    x = x_ref[...]
    cb = cb_ref[...]
    xc = lax.dot_general(x, cb, (((1,), (1,)), ((), ())),
                         preferred_element_type=jnp.float32)
    x2 = jnp.sum(x * x, axis=1, keepdims=True)
    d = x2 - 2.0 * xc + c2_ref[...]
    m = jnp.min(d, axis=1, keepdims=True)
    kio = lax.broadcasted_iota(jnp.int32, (BN, K), 1)
    a = jnp.min(jnp.where(d == m, kio, K), axis=1, keepdims=True)
    idx_ref[...] = bt_ref[...] * K + a
    loss_ref[...] += jnp.sum(m).reshape(1, 1)


_stage_a1 = pl.pallas_call(
    _stage_a1_body,
    grid=(NBLK,),
    in_specs=[
        pl.BlockSpec((BN, EMB), lambda i: (i, 0)),
        pl.BlockSpec((BN, 1), lambda i: (i, 0)),
        pl.BlockSpec((K, EMB), lambda i: (0, 0)),
    ],
    out_specs=[
        pl.BlockSpec((BN, 1), lambda i: (i, 0)),
        pl.BlockSpec((1, 1), lambda i: (0, 0)),
    ],
    out_shape=[
        jax.ShapeDtypeStruct((N, 1), jnp.int32),
        jax.ShapeDtypeStruct((1, 1), jnp.float32),
    ],
    scratch_shapes=[pltpu.VMEM((1, K), jnp.float32)],
    compiler_params=pltpu.CompilerParams(dimension_semantics=("arbitrary",)),
)


def _stage_a2_body(x_ref, sc_ref, bt_ref, p_ref, q_ref, cnt_ref):
    pid = pl.program_id(0)

    @pl.when(pid == 0)
    def _init():
        p_ref[...] = jnp.zeros_like(p_ref)
        q_ref[...] = jnp.zeros_like(q_ref)
        cnt_ref[...] = jnp.zeros_like(cnt_ref)

    x = x_ref[...]
    score = sc_ref[...]
    ohb = (lax.broadcasted_iota(jnp.int32, (BN, G), 1)
           == bt_ref[...]).astype(jnp.bfloat16)

    def tdot(lhs, rhs):
        return lax.dot_general(lhs, rhs, (((0,), (0,)), ((), ())),
                               preferred_element_type=jnp.float32)

    xh, xm = _split2(x)
    sh, sm = _split2(score)
    p_ref[...] += (tdot(ohb * sm, xh) + tdot(ohb * sh, xm)
                   + tdot(ohb * sh, xh))
    q_ref[...] += tdot(ohb, xm) + tdot(ohb, xh)
    cnt_ref[...] += tdot(ohb, jnp.ones((BN, 1), jnp.bfloat16))


_stage_a2 = pl.pallas_call(
    _stage_a2_body,
    grid=(NBLK,),
    in_specs=[
        pl.BlockSpec((BN, EMB), lambda i: (i, 0)),
        pl.BlockSpec((BN, 1), lambda i: (i, 0)),
        pl.BlockSpec((BN, 1), lambda i: (i, 0)),
    ],
    out_specs=[
        pl.BlockSpec((G, EMB), lambda i: (0, 0)),
        pl.BlockSpec((G, EMB), lambda i: (0, 0)),
        pl.BlockSpec((G, 1), lambda i: (0, 0)),
    ],
    out_shape=[
        jax.ShapeDtypeStruct((G, EMB), jnp.float32),
        jax.ShapeDtypeStruct((G, EMB), jnp.float32),
        jax.ShapeDtypeStruct((G, 1), jnp.float32),
    ],
    compiler_params=pltpu.CompilerParams(dimension_semantics=("arbitrary",)),
)


def _sc_stage_body(fl_hbm, sw_hbm, vw_hbm, out_hbm,
                   fi, fi2, sw, vw, zbuf, acc, sem, sem2):
    cid = lax.axis_index("c")
    sid = lax.axis_index("s")
    tid = cid * 16 + sid
    ld1 = pltpu.async_copy(fl_hbm.at[tid], fi, sem)
    ld2 = pltpu.async_copy(sw_hbm.at[tid], sw, sem)
    ld3 = pltpu.async_copy(vw_hbm.at[tid], vw, sem)
    for jj in range(ZB // 16):
        zbuf[pl.ds(jj * 16, 16)] = jnp.zeros((16,), jnp.float32)
    zc = [pltpu.async_copy(zbuf, acc.at[pl.ds(sid * STRIPE + kk * ZB, ZB)],
                           sem2)
          for kk in range(STRIPE // ZB)]
    ld1.wait()
    ld2.wait()
    ld3.wait()
    for j in range(CHUNKS):
        for l in range(CB // 16):
            s_ = pl.ds(l * 16, 16)
            fi2[j, s_] = fi[j, s_] + GK
    for c in zc:
        c.wait()
    plsc.subcore_barrier()
    for j in range(CHUNKS):
        c1 = pltpu.async_copy(sw.at[j], acc.at[fi.at[j]], sem, add=True)
        c2 = pltpu.async_copy(vw.at[j], acc.at[fi2.at[j]], sem2, add=True)
        c1.wait()
        c2.wait()
    plsc.subcore_barrier()
    pltpu.sync_copy(acc.at[pl.ds(sid * STRIPE, STRIPE)],
                    out_hbm.at[cid, pl.ds(sid * STRIPE, STRIPE)])


@functools.cache
def _build_sc_stage():
    return functools.partial(
        pl.kernel,
        mesh=plsc.VectorSubcoreMesh(core_axis_name="c", subcore_axis_name="s"),
        out_type=jax.ShapeDtypeStruct((2, 2 * GK), jnp.float32),
        scratch_types=[
            pltpu.VMEM((CHUNKS, CB), jnp.int32),
            pltpu.VMEM((CHUNKS, CB), jnp.int32),
            pltpu.VMEM((CHUNKS, CB), jnp.float32),
            pltpu.VMEM((CHUNKS, CB), jnp.float32),
            pltpu.VMEM((ZB,), jnp.float32),
            pltpu.VMEM_SHARED((2 * GK,), jnp.float32),
            pltpu.SemaphoreType.DMA,
            pltpu.SemaphoreType.DMA,
        ],
    )(_sc_stage_body)


def _stage_c_body(a_ref, cb_ref, p_ref, q_ref, cnt_ref, loss_ref, w_ref, b_ref,
                  logit_ref, cg_ref, sg_ref, lo_ref):
    a_s = a_ref[0, 0] + a_ref[1, 0]
    a_n = a_ref[0, 1] + a_ref[1, 1]
    cb = cb_ref[...]

    def ndot(lhs, rhs):
        return lax.dot_general(lhs, rhs, (((1,), (0,)), ((), ())),
                               preferred_element_type=jnp.float32)

    ch, cm = _split2(cb)
    sh, sm = _split2(a_s)
    nh, nm = _split2(a_n)
    r = ndot(sm, ch) + ndot(sh, cm) + ndot(sh, ch)
    s = ndot(nm, ch) + ndot(nh, cm) + ndot(nh, ch)
    cnt = jnp.maximum(cnt_ref[...], 1.0)
    p = p_ref[...]
    cr = p + r
    cg = cr / cnt
    sg = (q_ref[...] + s - cr) / cnt
    cg_ref[...] = cg
    sg_ref[...] = sg
    logit_ref[...] = lax.dot_general(cg, w_ref[...], (((1,), (0,)), ((), ())),
                                     preferred_element_type=jnp.float32) + b_ref[...]
    lo_ref[...] = loss_ref[...] * (CW / (N * EMB))


_stage_c = pl.pallas_call(
    _stage_c_body,
    out_shape=[
        jax.ShapeDtypeStruct((G, NC), jnp.float32),
        jax.ShapeDtypeStruct((G, EMB), jnp.float32),
        jax.ShapeDtypeStruct((G, EMB), jnp.float32),
        jax.ShapeDtypeStruct((1, 1), jnp.float32),
    ],
)


def kernel(node_feat, score, batch, codebook, W, b):
    batch = batch.astype(jnp.int32)

    fl, loss = _stage_a1(node_feat, batch[:, None], codebook)

    fl2 = jnp.zeros((N2,), jnp.int32).at[:N].set(fl[:, 0])
    sw2 = jnp.zeros((N2,), jnp.float32).at[:N].set(score[:, 0])
    vw2 = jnp.zeros((N2,), jnp.float32).at[:N].set(1.0)
    a_mats = _build_sc_stage()(fl2.reshape(NTILES, CHUNKS, CB),
                               sw2.reshape(NTILES, CHUNKS, CB),
                               vw2.reshape(NTILES, CHUNKS, CB))

    p_sum, q_sum, cnt = _stage_a2(node_feat, score, batch[:, None])

    logit, c_graph, s_graph, lo = _stage_c(
        a_mats.reshape(2, 2, G, K), codebook, p_sum, q_sum, cnt, loss,
        W, b[None, :])
    return (logit, c_graph, s_graph, lo[0, 0])


# speedup vs baseline: 1.4986x; 1.1992x over previous
import functools

import jax
import jax.numpy as jnp
from jax import lax
from jax.experimental import pallas as pl
from jax.experimental.pallas import tpu as pltpu
from jax.experimental.pallas import tpu_sc as plsc

N = 10000
EMB = 256
K = 1024
G = 128
NC = 10
CW = 1.0

BN = 1000
NBLK = N // BN

NTILES = 32
CHUNKS = 3
CB = 128
PER_TILE = CHUNKS * CB
N2 = NTILES * PER_TILE
GK = G * K
STRIPE = 2 * GK // 16
ZB = 2048


def _split3(v):
    h = v.astype(jnp.bfloat16)
    r = v - h.astype(jnp.float32)
    mid = r.astype(jnp.bfloat16)
    low = (r - mid.astype(jnp.float32)).astype(jnp.bfloat16)
    return h, mid, low


def _split2(v):
    h = v.astype(jnp.bfloat16)
    mid = (v - h.astype(jnp.float32)).astype(jnp.bfloat16)
    return h, mid


def _stage_a_body(x_ref, sc_ref, bt_ref, cb_ref,
                  fl_ref, p_ref, q_ref, cnt_ref, loss_ref, c2_ref):
    pid = pl.program_id(0)

    @pl.when(pid == 0)
    def _init():
        p_ref[...] = jnp.zeros_like(p_ref)
        q_ref[...] = jnp.zeros_like(q_ref)
        cnt_ref[...] = jnp.zeros_like(cnt_ref)
        loss_ref[...] = jnp.zeros_like(loss_ref)
        csh, csm, csl = _split3(cb_ref[...] * cb_ref[...])
        ones_col = jnp.ones((EMB, 1), jnp.bfloat16)

        def odot(lhs):
            return lax.dot_general(lhs, ones_col, (((1,), (0,)), ((), ())),
                                   preferred_element_type=jnp.float32)

        c2_ref[...] = odot(csl) + odot(csm) + odot(csh)

    x = x_ref[...]
    cb = cb_ref[...]
    xct = lax.dot_general(cb, x, (((1,), (1,)), ((), ())),
                          preferred_element_type=jnp.float32)
    dt = c2_ref[...] - 2.0 * xct
    m = jnp.min(dt, axis=0, keepdims=True)
    kio = lax.broadcasted_iota(jnp.int32, (K, BN), 0)
    a = jnp.min(jnp.where(dt == m, kio, K), axis=0, keepdims=True)

    bt = bt_ref[0]
    fl_ref[0] = bt * K + a

    xsq = x * x
    xsh, xsm = _split2(xsq)
    ones_colb = jnp.ones((EMB, 1), jnp.bfloat16)

    def xdot(lhs):
        return lax.dot_general(lhs, ones_colb, (((1,), (0,)), ((), ())),
                               preferred_element_type=jnp.float32)

    x2sum = jnp.sum(xdot(xsm) + xdot(xsh))
    loss_ref[...] += (jnp.sum(m) + x2sum).reshape(1, 1)

    score = sc_ref[0]
    oh = (lax.broadcasted_iota(jnp.int32, (G, BN), 0)
          == bt).astype(jnp.bfloat16)

    def ndot(lhs, rhs):
        return lax.dot_general(lhs, rhs, (((1,), (0,)), ((), ())),
                               preferred_element_type=jnp.float32)

    xh, xm = _split2(x)
    sh, sm = _split2(score)
    p_ref[...] += (ndot(oh * sm, xh) + ndot(oh * sh, xm)
                   + ndot(oh * sh, xh))
    q_ref[...] += ndot(oh, xm) + ndot(oh, xh)
    cnt_ref[...] += ndot(oh, jnp.ones((BN, 1), jnp.bfloat16))


_stage_a = pl.pallas_call(
    _stage_a_body,
    grid=(NBLK,),
    in_specs=[
        pl.BlockSpec((BN, EMB), lambda i: (i, 0)),
        pl.BlockSpec((1, 1, BN), lambda i: (i, 0, 0)),
        pl.BlockSpec((1, 1, BN), lambda i: (i, 0, 0)),
        pl.BlockSpec((K, EMB), lambda i: (0, 0)),
    ],
    out_specs=[
        pl.BlockSpec((1, 1, BN), lambda i: (i, 0, 0)),
        pl.BlockSpec((G, EMB), lambda i: (0, 0)),
        pl.BlockSpec((G, EMB), lambda i: (0, 0)),
        pl.BlockSpec((G, 1), lambda i: (0, 0)),
        pl.BlockSpec((1, 1), lambda i: (0, 0)),
    ],
    out_shape=[
        jax.ShapeDtypeStruct((NBLK, 1, BN), jnp.int32),
        jax.ShapeDtypeStruct((G, EMB), jnp.float32),
        jax.ShapeDtypeStruct((G, EMB), jnp.float32),
        jax.ShapeDtypeStruct((G, 1), jnp.float32),
        jax.ShapeDtypeStruct((1, 1), jnp.float32),
    ],
    scratch_shapes=[pltpu.VMEM((K, 1), jnp.float32)],
    compiler_params=pltpu.CompilerParams(dimension_semantics=("arbitrary",)),
)


def _sc_stage_body(fl_hbm, sw_hbm, vw_hbm, out_hbm,
                   fi, fi2, sw, vw, zbuf, acc, sem, sem2):
    cid = lax.axis_index("c")
    sid = lax.axis_index("s")
    tid = cid * 16 + sid
    ld1 = pltpu.async_copy(fl_hbm.at[tid], fi, sem)
    ld2 = pltpu.async_copy(sw_hbm.at[tid], sw, sem)
    ld3 = pltpu.async_copy(vw_hbm.at[tid], vw, sem)
    for jj in range(ZB // 16):
        zbuf[pl.ds(jj * 16, 16)] = jnp.zeros((16,), jnp.float32)
    zc = [pltpu.async_copy(zbuf, acc.at[pl.ds(sid * STRIPE + kk * ZB, ZB)],
                           sem2)
          for kk in range(STRIPE // ZB)]
    ld1.wait()
    ld2.wait()
    ld3.wait()
    for j in range(CHUNKS):
        for l in range(CB // 16):
            s_ = pl.ds(l * 16, 16)
            fi2[j, s_] = fi[j, s_] + GK
    for c in zc:
        c.wait()
    plsc.subcore_barrier()
    for j in range(CHUNKS):
        c1 = pltpu.async_copy(sw.at[j], acc.at[fi.at[j]], sem, add=True)
        c2 = pltpu.async_copy(vw.at[j], acc.at[fi2.at[j]], sem2, add=True)
        c1.wait()
        c2.wait()
    plsc.subcore_barrier()
    pltpu.sync_copy(acc.at[pl.ds(sid * STRIPE, STRIPE)],
                    out_hbm.at[cid, pl.ds(sid * STRIPE, STRIPE)])


@functools.cache
def _build_sc_stage():
    return functools.partial(
        pl.kernel,
        mesh=plsc.VectorSubcoreMesh(core_axis_name="c", subcore_axis_name="s"),
        out_type=jax.ShapeDtypeStruct((2, 2 * GK), jnp.float32),
        scratch_types=[
            pltpu.VMEM((CHUNKS, CB), jnp.int32),
            pltpu.VMEM((CHUNKS, CB), jnp.int32),
            pltpu.VMEM((CHUNKS, CB), jnp.float32),
            pltpu.VMEM((CHUNKS, CB), jnp.float32),
            pltpu.VMEM((ZB,), jnp.float32),
            pltpu.VMEM_SHARED((2 * GK,), jnp.float32),
            pltpu.SemaphoreType.DMA,
            pltpu.SemaphoreType.DMA,
        ],
    )(_sc_stage_body)


def _stage_c_body(a_ref, cb_ref, p_ref, q_ref, cnt_ref, loss_ref, w_ref, b_ref,
                  logit_ref, cg_ref, sg_ref, lo_ref):
    a_s = a_ref[0, 0] + a_ref[1, 0]
    a_n = a_ref[0, 1] + a_ref[1, 1]
    cb = cb_ref[...]

    def ndot(lhs, rhs):
        return lax.dot_general(lhs, rhs, (((1,), (0,)), ((), ())),
                               preferred_element_type=jnp.float32)

    ch, cm = _split2(cb)
    sh, sm = _split2(a_s)
    nh, nm = _split2(a_n)
    r = ndot(sm, ch) + ndot(sh, cm) + ndot(sh, ch)
    s = ndot(nm, ch) + ndot(nh, cm) + ndot(nh, ch)
    cnt = jnp.maximum(cnt_ref[...], 1.0)
    p = p_ref[...]
    cr = p + r
    cg = cr / cnt
    sg = (q_ref[...] + s - cr) / cnt
    cg_ref[...] = cg
    sg_ref[...] = sg
    logit_ref[...] = lax.dot_general(cg, w_ref[...], (((1,), (0,)), ((), ())),
                                     preferred_element_type=jnp.float32) + b_ref[...]
    lo_ref[...] = loss_ref[...] * (CW / (N * EMB))


_stage_c = pl.pallas_call(
    _stage_c_body,
    out_shape=[
        jax.ShapeDtypeStruct((G, NC), jnp.float32),
        jax.ShapeDtypeStruct((G, EMB), jnp.float32),
        jax.ShapeDtypeStruct((G, EMB), jnp.float32),
        jax.ShapeDtypeStruct((1, 1), jnp.float32),
    ],
)


def kernel(node_feat, score, batch, codebook, W, b):
    batch = batch.astype(jnp.int32)
    score_row = score[:, 0]

    fl, p_sum, q_sum, cnt, loss = _stage_a(
        node_feat, score_row.reshape(NBLK, 1, BN),
        batch.reshape(NBLK, 1, BN), codebook)

    fl2 = jnp.zeros((N2,), jnp.int32).at[:N].set(fl.reshape(N))
    sw2 = jnp.zeros((N2,), jnp.float32).at[:N].set(score_row)
    vw2 = jnp.zeros((N2,), jnp.float32).at[:N].set(1.0)
    a_mats = _build_sc_stage()(fl2.reshape(NTILES, CHUNKS, CB),
                               sw2.reshape(NTILES, CHUNKS, CB),
                               vw2.reshape(NTILES, CHUNKS, CB))

    logit, c_graph, s_graph, lo = _stage_c(
        a_mats.reshape(2, 2, G, K), codebook, p_sum, q_sum, cnt, loss,
        W, b[None, :])
    return (logit, c_graph, s_graph, lo[0, 0])


# speedup vs baseline: 1.5854x; 1.0579x over previous
import functools

import jax
import jax.numpy as jnp
from jax import lax
from jax.experimental import pallas as pl
from jax.experimental.pallas import tpu as pltpu
from jax.experimental.pallas import tpu_sc as plsc

N = 10000
EMB = 256
K = 1024
G = 128
NC = 10
CW = 1.0

BN = 1000
NBLK = N // BN

NTILES = 32
CHUNKS = 3
CB = 128
PER_TILE = CHUNKS * CB
N2 = NTILES * PER_TILE
GK = G * K
STRIPE = 2 * GK // 16
ZB = 2048


def _split3(v):
    h = v.astype(jnp.bfloat16)
    r = v - h.astype(jnp.float32)
    mid = r.astype(jnp.bfloat16)
    low = (r - mid.astype(jnp.float32)).astype(jnp.bfloat16)
    return h, mid, low


def _split2(v):
    h = v.astype(jnp.bfloat16)
    mid = (v - h.astype(jnp.float32)).astype(jnp.bfloat16)
    return h, mid


def _stage_a_body(x_ref, sc_ref, bt_ref, cb_ref,
                  fl_ref, p_ref, q_ref, cnt_ref, loss_ref, c2_ref):
    pid = pl.program_id(0)

    @pl.when(pid == 0)
    def _init():
        p_ref[...] = jnp.zeros_like(p_ref)
        q_ref[...] = jnp.zeros_like(q_ref)
        cnt_ref[...] = jnp.zeros_like(cnt_ref)
        loss_ref[...] = jnp.zeros_like(loss_ref)
        csh, csm, csl = _split3(cb_ref[...] * cb_ref[...])
        ones_row = jnp.ones((1, EMB), jnp.bfloat16)

        def odot(rhs):
            return lax.dot_general(ones_row, rhs, (((1,), (1,)), ((), ())),
                                   preferred_element_type=jnp.float32)

        c2_ref[...] = odot(csl) + odot(csm) + odot(csh)

    x = x_ref[...]
    cb = cb_ref[...]
    xc = lax.dot_general(x, cb, (((1,), (1,)), ((), ())),
                         preferred_element_type=jnp.float32)
    x2 = jnp.sum(x * x, axis=1, keepdims=True)
    d = x2 - 2.0 * xc + c2_ref[...]
    m = jnp.min(d, axis=1, keepdims=True)
    kio = lax.broadcasted_iota(jnp.int32, (BN, K), 1)
    a = jnp.min(jnp.where(d == m, kio, K), axis=1, keepdims=True)

    bt = bt_ref[0]
    fl_ref[0] = bt * K + jnp.transpose(a)
    loss_ref[...] += jnp.sum(m).reshape(1, 1)

    score = sc_ref[0]
    oh = (lax.broadcasted_iota(jnp.int32, (G, BN), 0)
          == bt).astype(jnp.bfloat16)

    def ndot(lhs, rhs):
        return lax.dot_general(lhs, rhs, (((1,), (0,)), ((), ())),
                               preferred_element_type=jnp.float32)

    xh, xm = _split2(x)
    sh, sm = _split2(score)
    p_ref[...] += (ndot(oh * sm, xh) + ndot(oh * sh, xm)
                   + ndot(oh * sh, xh))
    q_ref[...] += ndot(oh, xm) + ndot(oh, xh)
    cnt_ref[...] += ndot(oh, jnp.ones((BN, 1), jnp.bfloat16))


_stage_a = pl.pallas_call(
    _stage_a_body,
    grid=(NBLK,),
    in_specs=[
        pl.BlockSpec((BN, EMB), lambda i: (i, 0)),
        pl.BlockSpec((1, 1, BN), lambda i: (i, 0, 0)),
        pl.BlockSpec((1, 1, BN), lambda i: (i, 0, 0)),
        pl.BlockSpec((K, EMB), lambda i: (0, 0)),
    ],
    out_specs=[
        pl.BlockSpec((1, 1, BN), lambda i: (i, 0, 0)),
        pl.BlockSpec((G, EMB), lambda i: (0, 0)),
        pl.BlockSpec((G, EMB), lambda i: (0, 0)),
        pl.BlockSpec((G, 1), lambda i: (0, 0)),
        pl.BlockSpec((1, 1), lambda i: (0, 0)),
    ],
    out_shape=[
        jax.ShapeDtypeStruct((NBLK, 1, BN), jnp.int32),
        jax.ShapeDtypeStruct((G, EMB), jnp.float32),
        jax.ShapeDtypeStruct((G, EMB), jnp.float32),
        jax.ShapeDtypeStruct((G, 1), jnp.float32),
        jax.ShapeDtypeStruct((1, 1), jnp.float32),
    ],
    scratch_shapes=[pltpu.VMEM((1, K), jnp.float32)],
    compiler_params=pltpu.CompilerParams(dimension_semantics=("arbitrary",)),
)


def _sc_stage_body(fl_hbm, sw_hbm, vw_hbm, out_hbm,
                   fi, fi2, sw, vw, zbuf, acc, sem, sem2):
    cid = lax.axis_index("c")
    sid = lax.axis_index("s")
    tid = cid * 16 + sid
    ld1 = pltpu.async_copy(fl_hbm.at[tid], fi, sem)
    ld2 = pltpu.async_copy(sw_hbm.at[tid], sw, sem)
    ld3 = pltpu.async_copy(vw_hbm.at[tid], vw, sem)
    for jj in range(ZB // 16):
        zbuf[pl.ds(jj * 16, 16)] = jnp.zeros((16,), jnp.float32)
    zc = [pltpu.async_copy(zbuf, acc.at[pl.ds(sid * STRIPE + kk * ZB, ZB)],
                           sem2)
          for kk in range(STRIPE // ZB)]
    ld1.wait()
    ld2.wait()
    ld3.wait()
    for j in range(CHUNKS):
        for l in range(CB // 16):
            s_ = pl.ds(l * 16, 16)
            fi2[j, s_] = fi[j, s_] + GK
    for c in zc:
        c.wait()
    plsc.subcore_barrier()
    for j in range(CHUNKS):
        c1 = pltpu.async_copy(sw.at[j], acc.at[fi.at[j]], sem, add=True)
        c2 = pltpu.async_copy(vw.at[j], acc.at[fi2.at[j]], sem2, add=True)
        c1.wait()
        c2.wait()
    plsc.subcore_barrier()
    pltpu.sync_copy(acc.at[pl.ds(sid * STRIPE, STRIPE)],
                    out_hbm.at[cid, pl.ds(sid * STRIPE, STRIPE)])


@functools.cache
def _build_sc_stage():
    return functools.partial(
        pl.kernel,
        mesh=plsc.VectorSubcoreMesh(core_axis_name="c", subcore_axis_name="s"),
        out_type=jax.ShapeDtypeStruct((2, 2 * GK), jnp.float32),
        scratch_types=[
            pltpu.VMEM((CHUNKS, CB), jnp.int32),
            pltpu.VMEM((CHUNKS, CB), jnp.int32),
            pltpu.VMEM((CHUNKS, CB), jnp.float32),
            pltpu.VMEM((CHUNKS, CB), jnp.float32),
            pltpu.VMEM((ZB,), jnp.float32),
            pltpu.VMEM_SHARED((2 * GK,), jnp.float32),
            pltpu.SemaphoreType.DMA,
            pltpu.SemaphoreType.DMA,
        ],
    )(_sc_stage_body)


def _stage_c_body(a_ref, cb_ref, p_ref, q_ref, cnt_ref, loss_ref, w_ref, b_ref,
                  logit_ref, cg_ref, sg_ref, lo_ref):
    a_s = a_ref[0, 0] + a_ref[1, 0]
    a_n = a_ref[0, 1] + a_ref[1, 1]
    cb = cb_ref[...]

    def ndot(lhs, rhs):
        return lax.dot_general(lhs, rhs, (((1,), (0,)), ((), ())),
                               preferred_element_type=jnp.float32)

    ch, cm = _split2(cb)
    sh, sm = _split2(a_s)
    nh, nm = _split2(a_n)
    r = ndot(sm, ch) + ndot(sh, cm) + ndot(sh, ch)
    s = ndot(nm, ch) + ndot(nh, cm) + ndot(nh, ch)
    cnt = jnp.maximum(cnt_ref[...], 1.0)
    p = p_ref[...]
    cr = p + r
    cg = cr / cnt
    sg = (q_ref[...] + s - cr) / cnt
    cg_ref[...] = cg
    sg_ref[...] = sg
    logit_ref[...] = lax.dot_general(cg, w_ref[...], (((1,), (0,)), ((), ())),
                                     preferred_element_type=jnp.float32) + b_ref[...]
    lo_ref[...] = loss_ref[...] * (CW / (N * EMB))


_stage_c = pl.pallas_call(
    _stage_c_body,
    out_shape=[
        jax.ShapeDtypeStruct((G, NC), jnp.float32),
        jax.ShapeDtypeStruct((G, EMB), jnp.float32),
        jax.ShapeDtypeStruct((G, EMB), jnp.float32),
        jax.ShapeDtypeStruct((1, 1), jnp.float32),
    ],
)


def kernel(node_feat, score, batch, codebook, W, b):
    batch = batch.astype(jnp.int32)
    score_row = score[:, 0]

    fl, p_sum, q_sum, cnt, loss = _stage_a(
        node_feat, score_row.reshape(NBLK, 1, BN),
        batch.reshape(NBLK, 1, BN), codebook)

    fl2 = jnp.zeros((N2,), jnp.int32).at[:N].set(fl.reshape(N))
    sw2 = jnp.zeros((N2,), jnp.float32).at[:N].set(score_row)
    vw2 = jnp.zeros((N2,), jnp.float32).at[:N].set(1.0)
    a_mats = _build_sc_stage()(fl2.reshape(NTILES, CHUNKS, CB),
                               sw2.reshape(NTILES, CHUNKS, CB),
                               vw2.reshape(NTILES, CHUNKS, CB))

    logit, c_graph, s_graph, lo = _stage_c(
        a_mats.reshape(2, 2, G, K), codebook, p_sum, q_sum, cnt, loss,
        W, b[None, :])
    return (logit, c_graph, s_graph, lo[0, 0])
